# Initial kernel scaffold; baseline (speedup 1.0000x reference)
import jax
import jax.numpy as jnp
from jax.experimental import pallas as pl


def kernel(query, key, value, proj_dir):
    raise NotImplementedError("write your pallas kernel here")


# speedup vs baseline: 2.3409x; 2.3409x over previous
import functools
import math

import jax
import jax.numpy as jnp
import numpy as np
from jax import lax
from jax.experimental import pallas as pl
from jax.experimental.pallas import tpu as pltpu

NUM_PROJS = 7
NBUCKET = 128
BLK = 256
SAMP = 256


def _perm_np():
    a = np.array([0, 1], dtype=np.int32)
    for _ in range(NUM_PROJS - 1):
        a = np.concatenate([a, np.flip(a) + a.shape[0]], 0)
    return a


_PERM_NP = _perm_np()
_INVPERM_NP = np.argsort(_PERM_NP).astype(np.float32)


def _positions_body(q_ref, k_ref, proj_ref, w_ref, invp_ref, m128_ref, t256_ref,
                    posq_ref, posk_ref, oh_ref, carr_ref):
    S = q_ref.shape[1]
    nchunk = S // BLK
    base = (pl.program_id(0) * S).astype(jnp.float32)

    def one(mat_ref, out_ref):
        def pass1(c, hist):
            rows = mat_ref[0, pl.ds(c * BLK, BLK), :]
            p = jnp.dot(rows, proj_ref[...], preferred_element_type=jnp.float32)
            binf = jnp.sum(jnp.where(p > 0, w_ref[...], 0.0), axis=1,
                           keepdims=True)
            oh = (binf == invp_ref[...]).astype(jnp.float32)
            carr_ref[pl.ds(c, 1), :] = hist
            oh_ref[pl.ds(c * BLK, BLK), :] = oh
            return hist + jnp.sum(oh, axis=0, keepdims=True)

        hist = lax.fori_loop(0, nchunk, pass1,
                             jnp.zeros((1, NBUCKET), jnp.float32))
        offs = jnp.dot(hist, m128_ref[...], preferred_element_type=jnp.float32)

        def pass2(c, _):
            oh = oh_ref[pl.ds(c * BLK, BLK), :]
            pe = jnp.dot(t256_ref[...], oh, preferred_element_type=jnp.float32)
            tot = pe + carr_ref[pl.ds(c, 1), :] + offs
            posf = jnp.sum(oh * tot, axis=1, keepdims=True) + base
            out_ref[0, pl.ds(c * BLK, BLK), :] = posf.astype(jnp.int32)
            return 0

        lax.fori_loop(0, nchunk, pass2, 0)

    one(q_ref, posq_ref)
    one(k_ref, posk_ref)


def _positions(q3, k3, proj):
    BH, S, D = q3.shape
    projpad = jnp.pad(proj.astype(jnp.float32), ((0, 0), (0, NBUCKET - NUM_PROJS)))
    w = jnp.asarray(
        np.concatenate([2.0 ** np.arange(NUM_PROJS),
                        np.zeros(NBUCKET - NUM_PROJS)]).astype(np.float32)
    ).reshape(1, NBUCKET)
    invp = jnp.asarray(_INVPERM_NP).reshape(1, NBUCKET)
    m128 = jnp.asarray(
        np.triu(np.ones((NBUCKET, NBUCKET), np.float32), 1))
    t256 = jnp.asarray(np.tril(np.ones((BLK, BLK), np.float32), -1))

    grid = (BH,)
    out = pl.pallas_call(
        _positions_body,
        grid=grid,
        in_specs=[
            pl.BlockSpec((1, S, D), lambda i: (i, 0, 0)),
            pl.BlockSpec((1, S, D), lambda i: (i, 0, 0)),
            pl.BlockSpec((D, NBUCKET), lambda i: (0, 0)),
            pl.BlockSpec((1, NBUCKET), lambda i: (0, 0)),
            pl.BlockSpec((1, NBUCKET), lambda i: (0, 0)),
            pl.BlockSpec((NBUCKET, NBUCKET), lambda i: (0, 0)),
            pl.BlockSpec((BLK, BLK), lambda i: (0, 0)),
        ],
        out_specs=[
            pl.BlockSpec((1, S, 1), lambda i: (i, 0, 0)),
            pl.BlockSpec((1, S, 1), lambda i: (i, 0, 0)),
        ],
        out_shape=[
            jax.ShapeDtypeStruct((BH, S, 1), jnp.int32),
            jax.ShapeDtypeStruct((BH, S, 1), jnp.int32),
        ],
        scratch_shapes=[
            pltpu.VMEM((S, NBUCKET), jnp.float32),
            pltpu.VMEM((S // BLK, NBUCKET), jnp.float32),
        ],
    )(q3, k3, projpad, w, invp, m128, t256)
    return out


def _attention_body(q_ref, k_ref, v_ref, ks_ref, vs_ref, sd_ref, out_ref,
                    *, scale, logw):
    q = q_ref[0, 0]
    kb = k_ref[0, 0]
    vb = v_ref[0, 0]
    ksp = ks_ref[0]
    vsp = vs_ref[0]
    sd = sd_ref[0]
    blk = pl.program_id(1)

    nt = (((1,), (1,)), ((), ()))
    lb = lax.dot_general(q, kb, nt, preferred_element_type=jnp.float32) * scale
    ls = lax.dot_general(q, ksp, nt, preferred_element_type=jnp.float32) * scale
    bias = jnp.where(sd == blk, jnp.float32(np.finfo(np.float32).min),
                     jnp.float32(logw))
    ls = ls + bias
    l = jnp.concatenate([lb, ls], axis=1)
    m = jnp.max(l, axis=1, keepdims=True)
    p = jnp.exp(l - m)
    s = jnp.sum(p, axis=1, keepdims=True)
    vall = jnp.concatenate([vb, vsp], axis=0)
    o = jnp.dot(p, vall, preferred_element_type=jnp.float32) / s
    out_ref[0, 0] = o


def _attention(qs4, ks4, vs4, ksamp, vsamp, ssdiv, scale, logw):
    BH, NB, _, D = qs4.shape
    body = functools.partial(_attention_body, scale=scale, logw=logw)
    return pl.pallas_call(
        body,
        grid=(BH, NB),
        in_specs=[
            pl.BlockSpec((1, 1, BLK, D), lambda i, j: (i, j, 0, 0)),
            pl.BlockSpec((1, 1, BLK, D), lambda i, j: (i, j, 0, 0)),
            pl.BlockSpec((1, 1, BLK, D), lambda i, j: (i, j, 0, 0)),
            pl.BlockSpec((1, SAMP, D), lambda i, j: (i, 0, 0)),
            pl.BlockSpec((1, SAMP, D), lambda i, j: (i, 0, 0)),
            pl.BlockSpec((1, 1, SAMP), lambda i, j: (i, 0, 0)),
        ],
        out_specs=pl.BlockSpec((1, 1, BLK, D), lambda i, j: (i, j, 0, 0)),
        out_shape=jax.ShapeDtypeStruct((BH, NB, BLK, D), jnp.float32),
    )(qs4, ks4, vs4, ksamp, vsamp, ssdiv)


def kernel(query, key, value, proj_dir):
    B, H, S, D = query.shape
    BH = B * H
    NB = S // BLK
    scale = D ** (-0.5)
    logw = math.log(S / SAMP)

    q3 = query.reshape(BH, S, D)
    k3 = key.reshape(BH, S, D)
    v3 = value.reshape(BH, S, D)

    posq3, posk3 = _positions(q3, k3, proj_dir)
    posq = posq3.reshape(BH * S)
    posk = posk3.reshape(BH * S)

    skey = jax.random.key(42)
    sampled = jax.random.randint(skey, (B, H, SAMP), 0, S)
    ssdiv = (sampled // BLK).astype(jnp.int32).reshape(BH, 1, SAMP)
    ssflat = (sampled.reshape(BH, SAMP)
              + jnp.arange(BH, dtype=jnp.int32)[:, None] * S).reshape(-1)

    q2 = q3.reshape(BH * S, D)
    k2 = k3.reshape(BH * S, D)
    v2 = v3.reshape(BH * S, D)
    qs2 = jnp.zeros_like(q2).at[posq].set(q2)
    ks2 = jnp.zeros_like(k2).at[posk].set(k2)
    vs2 = jnp.zeros_like(v2).at[posk].set(v2)
    ksamp = ks2[ssflat].reshape(BH, SAMP, D)
    vsamp = vs2[ssflat].reshape(BH, SAMP, D)

    attn4 = _attention(qs2.reshape(BH, NB, BLK, D), ks2.reshape(BH, NB, BLK, D),
                       vs2.reshape(BH, NB, BLK, D), ksamp, vsamp, ssdiv,
                       scale, logw)
    attn2 = attn4.reshape(BH * S, D)
    out2 = attn2[posq]
    return out2.reshape(B, H, S, D)
